# Initial kernel scaffold; baseline (speedup 1.0000x reference)
import jax
import jax.numpy as jnp
from jax.experimental import pallas as pl


def kernel(drug_feat, target_feat, disease_feat, ei_targets, ei_targeted_by, ei_similar_to, ei_treats, ei_treated_by, ei_associated_with, ei_has_target, params):
    raise NotImplementedError("write your pallas kernel here")


# speedup vs baseline: 1.9321x; 1.9321x over previous
import functools
import math

import jax
import jax.numpy as jnp
from jax import lax
from jax.experimental import pallas as pl
from jax.experimental.pallas import tpu as pltpu
from jax.experimental.pallas import tpu_sc as plsc

_ETYPES = [
    ("drug", "targets", "target"),
    ("target", "targeted_by", "drug"),
    ("drug", "similar_to", "drug"),
    ("drug", "treats", "disease"),
    ("disease", "treated_by", "drug"),
    ("target", "associated_with", "disease"),
    ("disease", "has_target", "target"),
]
_NTYPES = ["drug", "target", "disease"]
_N = 10000
_E = 64000
_HID = 256
_HALF = 128
_CHUNK = 64
_ROWS = 1024
_EPAD = _ROWS * _CHUNK - _E
_TRASH = _N
_NSC = 2
_NSUB = 16
_RPT = _ROWS // _NSUB
_ACCR = 10112
_SLAB = _ACCR // _NSUB
_BN = 1.0 / math.sqrt(1.0 + 1e-5)
_F32 = jnp.float32


_ZROWS = 64


def _slab_chunks():
    out, off = [], 0
    while off < _SLAB:
        sz = min(_ZROWS, _SLAB - off)
        out.append((off, sz))
        off += sz
    return out


def _make_segsum(with_counts=False):
    del with_counts
    mesh = plsc.VectorSubcoreMesh(core_axis_name="c", subcore_axis_name="s",
                                  num_cores=_NSC, num_subcores=_NSUB)
    out_type = jax.ShapeDtypeStruct((7, _NSC, _ACCR, _HALF), _F32)
    scratch = [
        pltpu.VMEM((_RPT, _CHUNK), jnp.int32),
        pltpu.VMEM((_RPT, _CHUNK), jnp.int32),
        pltpu.VMEM((_CHUNK, _HALF), _F32),
        pltpu.VMEM((_ZROWS, _HALF), _F32),
        pltpu.VMEM_SHARED((_ACCR, _HALF), _F32),
        pltpu.SemaphoreType.DMA,
    ]

    def body(ht_drug, ht_target, ht_disease, srcidx, dstidx, z128_hbm,
             s_out, src_v, dst_v, rows_v, zbuf, acc, sem):
        hts = {"drug": ht_drug, "target": ht_target, "disease": ht_disease}
        c = lax.axis_index("c")
        s = lax.axis_index("s")
        row0 = s * _SLAB
        pltpu.sync_copy(z128_hbm, zbuf)
        for e, (sn, en, dn) in enumerate(_ETYPES):
            for off, sz in _slab_chunks():
                pltpu.sync_copy(zbuf.at[pl.ds(0, sz)],
                                acc.at[pl.ds(row0 + off, sz)])
            plsc.subcore_barrier()
            pltpu.sync_copy(srcidx.at[e, pl.ds(c * _ROWS + s * _RPT, _RPT)],
                            src_v)
            pltpu.sync_copy(dstidx.at[e, pl.ds(s * _RPT, _RPT)], dst_v)
            ht = hts[sn]

            def chunk(j, carry):
                pltpu.async_copy(ht.at[src_v.at[j]], rows_v, sem).wait()
                pltpu.sync_copy(rows_v, acc.at[dst_v.at[j]], add=True)
                return carry

            lax.fori_loop(0, _RPT, chunk, 0)
            plsc.subcore_barrier()
            pltpu.sync_copy(acc.at[pl.ds(row0, _SLAB)],
                            s_out.at[e, c, pl.ds(row0, _SLAB)])

    return pl.kernel(body, out_type=out_type, mesh=mesh,
                     scratch_types=scratch)


def _make_counts():
    mesh = plsc.VectorSubcoreMesh(core_axis_name="c", subcore_axis_name="s",
                                  num_cores=_NSC, num_subcores=_NSUB)
    out_type = jax.ShapeDtypeStruct((7, _ACCR, _HALF), _F32)
    scratch = [
        pltpu.VMEM((_RPT, _CHUNK), jnp.int32),
        pltpu.VMEM((_CHUNK, _HALF), _F32),
        pltpu.VMEM((_ZROWS, _HALF), _F32),
        pltpu.VMEM_SHARED((_ACCR, _HALF), _F32),
    ]

    def body(dstidx, z128_hbm, o128_hbm, cnt_out, dst_v, ones_v, zbuf,
             cnt_acc):
        c = lax.axis_index("c")
        s = lax.axis_index("s")
        row0 = s * _SLAB
        pltpu.sync_copy(z128_hbm, zbuf)
        pltpu.sync_copy(o128_hbm, ones_v)
        for e in range(7):
            owner = 0 if e < 4 else 1

            @pl.when(c == owner)
            def _do(e=e):
                for off, sz in _slab_chunks():
                    pltpu.sync_copy(zbuf.at[pl.ds(0, sz)],
                                    cnt_acc.at[pl.ds(row0 + off, sz)])
                plsc.subcore_barrier()
                pltpu.sync_copy(dstidx.at[e, pl.ds(s * _RPT, _RPT)], dst_v)

                def chunk(j, carry):
                    pltpu.sync_copy(ones_v, cnt_acc.at[dst_v.at[j]], add=True)
                    return carry

                lax.fori_loop(0, _RPT, chunk, 0)
                plsc.subcore_barrier()
                pltpu.sync_copy(cnt_acc.at[pl.ds(row0, _SLAB)],
                                cnt_out.at[e, pl.ds(row0, _SLAB)])

    return pl.kernel(body, out_type=out_type, mesh=mesh,
                     scratch_types=scratch)


def _dense(xs, ws, cnt_ids, cnts, bias, outd, do_relu, post_scale):
    n, r = _N, 400
    grid = (n // r,)
    nx, nc = len(xs), len(cnts)

    def body(*refs):
        xrefs = refs[:nx]
        wrefs = refs[nx:2 * nx]
        crefs = refs[2 * nx:2 * nx + nc]
        bref = refs[2 * nx + nc]
        oref = refs[-1]
        recips = [1.0 / jnp.maximum(cr[...][:, :1], 1.0) for cr in crefs]
        acc = jnp.zeros((r, outd), _F32)
        for i in range(nx):
            x = xrefs[i][...]
            if cnt_ids[i] is not None:
                x = x * recips[cnt_ids[i]]
            acc = acc + jnp.dot(x, wrefs[i][...],
                                preferred_element_type=_F32)
        acc = (acc + bref[...]) * post_scale
        if do_relu:
            acc = jnp.maximum(acc, 0.0)
        oref[...] = acc

    in_specs = (
        [pl.BlockSpec((r, x.shape[1]), lambda i: (i, 0)) for x in xs]
        + [pl.BlockSpec(w.shape, lambda i: (0, 0)) for w in ws]
        + [pl.BlockSpec((r, _HALF), lambda i: (i, 0)) for _ in cnts]
        + [pl.BlockSpec(bias.shape, lambda i: (0, 0))]
    )
    return pl.pallas_call(
        body, grid=grid, in_specs=in_specs,
        out_specs=pl.BlockSpec((r, outd), lambda i: (i, 0)),
        out_shape=jax.ShapeDtypeStruct((n, outd), _F32),
    )(*xs, *ws, *cnts, bias)


def _stack_halves(h):
    return h.reshape(_N, 2, _HALF).swapaxes(0, 1).reshape(2 * _N, _HALF)


def _layer(i, h, s_all, cnts, params, outd, do_relu):
    new = {}
    for nt in _NTYPES:
        es = [e for e, (sn, en, dn) in enumerate(_ETYPES) if dn == nt]
        enames = [_ETYPES[e][1] for e in es]
        wself = sum(params["Wself_%d_%s" % (i, en)] for en in enames)
        bias = sum(params["b_%d_%s" % (i, en)] for en in enames)
        xs, ws, cnt_ids, ccols = [h[nt]], [wself], [None], []
        for k, e in enumerate(es):
            wn = params["Wneigh_%d_%s" % (i, enames[k])]
            xs += [s_all[e, 0], s_all[e, 1]]
            ws += [wn[:_HALF], wn[_HALF:]]
            cnt_ids += [k, k]
            ccols.append(cnts[e])
        new[nt] = _dense(xs, ws, cnt_ids, ccols, bias.reshape(1, -1),
                         outd, do_relu, _BN)
    return new


def kernel(drug_feat, target_feat, disease_feat, ei_targets, ei_targeted_by,
           ei_similar_to, ei_treats, ei_treated_by, ei_associated_with,
           ei_has_target, params):
    eis = {"targets": ei_targets, "targeted_by": ei_targeted_by,
           "similar_to": ei_similar_to, "treats": ei_treats,
           "treated_by": ei_treated_by, "associated_with": ei_associated_with,
           "has_target": ei_has_target}
    feats = {"drug": drug_feat, "target": target_feat, "disease": disease_feat}

    srcs, dsts = [], []
    for (sn, en, dn) in _ETYPES:
        ei = eis[en]
        src = jnp.concatenate([ei[0], jnp.zeros((_EPAD,), jnp.int32)])
        dst = jnp.concatenate([ei[1], jnp.full((_EPAD,), _TRASH, jnp.int32)])
        srcs.append(jnp.concatenate([src, src + _N]).reshape(2 * _ROWS, _CHUNK))
        dsts.append(dst.reshape(_ROWS, _CHUNK))
    srcidx = jnp.stack(srcs)
    dstidx = jnp.stack(dsts)
    z128 = jnp.zeros((_ZROWS, _HALF), _F32)
    o128 = jnp.ones((_CHUNK, _HALF), _F32)

    h = {nt: _dense([feats[nt]], [params["Win_" + nt]], [None], [],
                    params["bin_" + nt].reshape(1, -1), _HID, True, 1.0)
         for nt in _NTYPES}

    ht = {nt: _stack_halves(h[nt]) for nt in _NTYPES}
    cnt_all = _make_counts()(dstidx, z128, o128)
    s0 = _make_segsum()(ht["drug"], ht["target"], ht["disease"],
                        srcidx, dstidx, z128)
    cnts = [cnt_all[e] for e in range(7)]
    h = _layer(0, h, s0, cnts, params, _HID, True)

    ht = {nt: _stack_halves(h[nt]) for nt in _NTYPES}
    s1 = _make_segsum()(ht["drug"], ht["target"], ht["disease"],
                        srcidx, dstidx, z128)
    h = _layer(1, h, s1, cnts, params, 128, False)

    return (h["drug"], h["target"], h["disease"])
